# baseline (speedup 1.0000x reference)
import functools

import jax
import jax.numpy as jnp
from jax.experimental import pallas as pl
from jax.experimental.pallas import tpu as pltpu

DM = 1024
NE = 16
ES = 128
TB = 512


def _prep_body(k_ref, v_ref, km_ref, vm_ref):
    km_ref[...] = k_ref[0].astype(jnp.bfloat16)
    vm_ref[...] = v_ref[0].astype(jnp.bfloat16)


def _prep_weights(keys, values):
    return pl.pallas_call(
        _prep_body,
        grid=(NE,),
        in_specs=[
            pl.BlockSpec((1, DM, ES), lambda e: (e, 0, 0)),
            pl.BlockSpec((1, ES, DM), lambda e: (e, 0, 0)),
        ],
        out_specs=[
            pl.BlockSpec((DM, ES), lambda e: (0, e)),
            pl.BlockSpec((ES, DM), lambda e: (e, 0)),
        ],
        out_shape=[
            jax.ShapeDtypeStruct((DM, NE * ES), jnp.bfloat16),
            jax.ShapeDtypeStruct((NE * ES, DM), jnp.bfloat16),
        ],
    )(keys, values)


def _moe_body(x_ref, wgt_ref, k_ref, v_ref, o_ref):
    xb = x_ref[...]
    logits = jnp.dot(xb, wgt_ref[...],
                     preferred_element_type=jnp.float32)
    sel = jax.nn.sigmoid(logits)
    lane = jax.lax.broadcasted_iota(jnp.int32, (TB, NE), 1)
    m1 = jnp.max(logits, axis=1, keepdims=True)
    a1 = jnp.min(jnp.where(logits == m1, lane, NE), axis=1, keepdims=True)
    hot1 = lane == a1
    l2 = jnp.where(hot1, -jnp.inf, logits)
    m2 = jnp.max(l2, axis=1, keepdims=True)
    a2 = jnp.min(jnp.where(l2 == m2, lane, NE), axis=1, keepdims=True)
    gate = sel * (hot1 | (lane == a2)).astype(jnp.float32)
    scores = jnp.dot(xb.astype(jnp.bfloat16), k_ref[...],
                     preferred_element_type=jnp.float32)
    h = jnp.concatenate(
        [jnp.maximum(scores[:, e * ES:(e + 1) * ES], 0.0) * gate[:, e:e + 1]
         for e in range(NE)], axis=1)
    o_ref[...] = jnp.dot(h.astype(jnp.bfloat16), v_ref[...],
                         preferred_element_type=jnp.float32)


@jax.jit
def kernel(x, w_gate, keys, values):
    B, S, D = x.shape
    xf = x.reshape(-1, D)
    n = xf.shape[0]
    kmat, vmat = _prep_weights(keys, values)
    wgt = w_gate.T
    grid = (n // TB,)
    out = pl.pallas_call(
        _moe_body,
        grid=grid,
        in_specs=[
            pl.BlockSpec((TB, D), lambda i: (i, 0)),
            pl.BlockSpec((D, NE), lambda i: (0, 0)),
            pl.BlockSpec((D, NE * ES), lambda i: (0, 0)),
            pl.BlockSpec((NE * ES, D), lambda i: (0, 0)),
        ],
        out_specs=pl.BlockSpec((TB, D), lambda i: (i, 0)),
        out_shape=jax.ShapeDtypeStruct((n, D), jnp.float32),
        compiler_params=pltpu.CompilerParams(
            dimension_semantics=("parallel",),
        ),
    )(xf, wgt, kmat, vmat)
    return out.reshape(B, S, D)


# speedup vs baseline: 1.3604x; 1.3604x over previous
import functools

import jax
import jax.numpy as jnp
from jax.experimental import pallas as pl
from jax.experimental.pallas import tpu as pltpu

DM = 1024
NE = 16
ES = 128
TB = 512


def _moe_body(x_ref, wgt_ref, k_ref, v_ref, o_ref, km_ref, vm_ref):
    @pl.when(pl.program_id(0) == 0)
    def _():
        for e in range(NE):
            km_ref[:, e * ES:(e + 1) * ES] = k_ref[e].astype(jnp.bfloat16)
            vm_ref[e * ES:(e + 1) * ES, :] = v_ref[e].astype(jnp.bfloat16)

    xb = x_ref[...]
    logits = jnp.dot(xb, wgt_ref[...],
                     preferred_element_type=jnp.float32)
    sel = jax.nn.sigmoid(logits)
    lane = jax.lax.broadcasted_iota(jnp.int32, (TB, NE), 1)
    m1 = jnp.max(logits, axis=1, keepdims=True)
    a1 = jnp.min(jnp.where(logits == m1, lane, NE), axis=1, keepdims=True)
    hot1 = lane == a1
    l2 = jnp.where(hot1, -jnp.inf, logits)
    m2 = jnp.max(l2, axis=1, keepdims=True)
    a2 = jnp.min(jnp.where(l2 == m2, lane, NE), axis=1, keepdims=True)
    gate = sel * (hot1 | (lane == a2)).astype(jnp.float32)
    scores = jnp.dot(xb.astype(jnp.bfloat16), km_ref[...],
                     preferred_element_type=jnp.float32)
    h = jnp.concatenate(
        [jnp.maximum(scores[:, e * ES:(e + 1) * ES], 0.0) * gate[:, e:e + 1]
         for e in range(NE)], axis=1)
    o_ref[...] = jnp.dot(h.astype(jnp.bfloat16), vm_ref[...],
                         preferred_element_type=jnp.float32)


@jax.jit
def kernel(x, w_gate, keys, values):
    B, S, D = x.shape
    xf = x.reshape(-1, D)
    n = xf.shape[0]
    wgt = w_gate.T
    grid = (n // TB,)
    out = pl.pallas_call(
        _moe_body,
        grid=grid,
        in_specs=[
            pl.BlockSpec((TB, D), lambda i: (i, 0)),
            pl.BlockSpec((D, NE), lambda i: (0, 0)),
            pl.BlockSpec((NE, D, ES), lambda i: (0, 0, 0)),
            pl.BlockSpec((NE, ES, D), lambda i: (0, 0, 0)),
        ],
        out_specs=pl.BlockSpec((TB, D), lambda i: (i, 0)),
        out_shape=jax.ShapeDtypeStruct((n, D), jnp.float32),
        scratch_shapes=[
            pltpu.VMEM((D, NE * ES), jnp.bfloat16),
            pltpu.VMEM((NE * ES, D), jnp.bfloat16),
        ],
        compiler_params=pltpu.CompilerParams(
            dimension_semantics=("arbitrary",),
        ),
    )(xf, wgt, keys, values)
    return out.reshape(B, S, D)


# speedup vs baseline: 1.4379x; 1.0569x over previous
import functools

import jax
import jax.numpy as jnp
from jax.experimental import pallas as pl
from jax.experimental.pallas import tpu as pltpu

DM = 1024
NE = 16
ES = 128
TB = 512


def _moe_body(x_ref, wgt_ref, k_ref, v_ref, o_ref, km_ref, vm_ref):
    @pl.when(pl.program_id(0) == 0)
    def _():
        for e in range(NE):
            km_ref[:, e * ES:(e + 1) * ES] = k_ref[e].astype(jnp.bfloat16)
            vm_ref[e * ES:(e + 1) * ES, :] = v_ref[e].astype(jnp.bfloat16)

    xb = x_ref[...]
    logits = jax.lax.dot_general(
        xb, wgt_ref[...], (((1,), (1,)), ((), ())),
        preferred_element_type=jnp.float32)
    sel = jax.nn.sigmoid(logits)
    lane = jax.lax.broadcasted_iota(jnp.int32, (TB, NE), 1)
    m1 = jnp.max(logits, axis=1, keepdims=True)
    a1 = jnp.min(jnp.where(logits == m1, lane, NE), axis=1, keepdims=True)
    hot1 = lane == a1
    l2 = jnp.where(hot1, -jnp.inf, logits)
    m2 = jnp.max(l2, axis=1, keepdims=True)
    a2 = jnp.min(jnp.where(l2 == m2, lane, NE), axis=1, keepdims=True)
    gate = sel * (hot1 | (lane == a2)).astype(jnp.float32)
    scores = jnp.dot(xb.astype(jnp.bfloat16), km_ref[...],
                     preferred_element_type=jnp.float32)
    h = jnp.concatenate(
        [jnp.maximum(scores[:, e * ES:(e + 1) * ES], 0.0) * gate[:, e:e + 1]
         for e in range(NE)], axis=1)
    o_ref[...] = jnp.dot(h.astype(jnp.bfloat16), vm_ref[...],
                         preferred_element_type=jnp.float32)


@jax.jit
def kernel(x, w_gate, keys, values):
    B, S, D = x.shape
    xf = x.reshape(-1, D)
    n = xf.shape[0]
    grid = (n // TB,)
    out = pl.pallas_call(
        _moe_body,
        grid=grid,
        in_specs=[
            pl.BlockSpec((TB, D), lambda i: (i, 0)),
            pl.BlockSpec((NE, D), lambda i: (0, 0)),
            pl.BlockSpec((NE, D, ES), lambda i: (0, 0, 0)),
            pl.BlockSpec((NE, ES, D), lambda i: (0, 0, 0)),
        ],
        out_specs=pl.BlockSpec((TB, D), lambda i: (i, 0)),
        out_shape=jax.ShapeDtypeStruct((n, D), jnp.float32),
        scratch_shapes=[
            pltpu.VMEM((D, NE * ES), jnp.bfloat16),
            pltpu.VMEM((NE * ES, D), jnp.bfloat16),
        ],
        compiler_params=pltpu.CompilerParams(
            dimension_semantics=("arbitrary",),
        ),
    )(xf, w_gate, keys, values)
    return out.reshape(B, S, D)
